# Initial kernel scaffold; baseline (speedup 1.0000x reference)
import jax
import jax.numpy as jnp
from jax.experimental import pallas as pl


def kernel(rays_o, rays_d, bg_color, grid0, features, w_s1, b_s1, w_s2, b_s2, w_c1, b_c1, w_c2, b_c2):
    raise NotImplementedError("write your pallas kernel here")


# speedup vs baseline: 38.3757x; 38.3757x over previous
import functools

import jax
import jax.numpy as jnp
from jax import lax
from jax.experimental import pallas as pl
from jax.experimental.pallas import tpu as pltpu
from jax.experimental.pallas import tpu_sc as plsc

N_RAYS = 2048
N_SAMPLES = 128
N_PTS = N_RAYS * N_SAMPLES
RADIUS = 1.5
RES_P = 128
RES_F = 64
FEAT = 32
OUT_DIM = 3
RANK = 4
WIDTH = 64

NW = 32
P_PER_W = N_PTS // NW
CHUNK = 128
N_CHUNK = P_PER_W // CHUNK
LANES = 16
GRP = CHUNK // LANES

RB = 128
N_RB = N_RAYS // RB


def _ray_setup_body(ro_ref, rd_ref, px_ref, py_ref, pz_ref, d_ref, dl_ref):
    ro = ro_ref[...]
    rd = rd_ref[...]
    nrm = jnp.sqrt(jnp.sum(rd * rd, axis=0, keepdims=True))
    d = rd / nrm
    b = jnp.sum(ro * d, axis=0)
    c = jnp.sum(ro * ro, axis=0) - RADIUS * RADIUS
    sq = jnp.sqrt(jnp.maximum(b * b - c, 1e-12))
    tn = jnp.maximum(-b - sq, 0.0)
    tf = jnp.maximum(-b + sq, tn + 1e-6)
    steps = (lax.broadcasted_iota(jnp.int32, (1, N_SAMPLES), 1).astype(jnp.float32)
             + 0.5) / N_SAMPLES
    t = tn[:, None] + (tf - tn)[:, None] * steps
    px_ref[...] = (ro[0][:, None] + t * d[0][:, None]) / RADIUS
    py_ref[...] = (ro[1][:, None] + t * d[1][:, None]) / RADIUS
    pz_ref[...] = (ro[2][:, None] + t * d[2][:, None]) / RADIUS
    d_ref[...] = d
    dl_ref[...] = ((tf - tn) / N_SAMPLES)[None, :]


_ray_setup = pl.pallas_call(
    _ray_setup_body,
    out_shape=(
        jax.ShapeDtypeStruct((N_RAYS, N_SAMPLES), jnp.float32),
        jax.ShapeDtypeStruct((N_RAYS, N_SAMPLES), jnp.float32),
        jax.ShapeDtypeStruct((N_RAYS, N_SAMPLES), jnp.float32),
        jax.ShapeDtypeStruct((3, N_RAYS), jnp.float32),
        jax.ShapeDtypeStruct((1, N_RAYS), jnp.float32),
    ),
)


_SC_MESH = plsc.VectorSubcoreMesh(core_axis_name="c", subcore_axis_name="s")


@functools.partial(
    pl.kernel,
    out_type=jax.ShapeDtypeStruct((N_PTS, FEAT), jnp.float32),
    mesh=_SC_MESH,
    compiler_params=pltpu.CompilerParams(use_tc_tiling_on_sc=False),
    scratch_types=[
        pltpu.VMEM((3, CHUNK), jnp.float32),
        pltpu.VMEM((12, CHUNK), jnp.int32),
        pltpu.VMEM((12, CHUNK), jnp.float32),
    ] + [pltpu.VMEM((CHUNK, 16), jnp.float32) for _ in range(12)] + [
        pltpu.VMEM((8, CHUNK), jnp.int32),
        pltpu.VMEM((8, CHUNK), jnp.float32),
    ] + [pltpu.VMEM((CHUNK, FEAT), jnp.float32) for _ in range(8)] + [
        pltpu.VMEM((CHUNK, FEAT), jnp.float32),
        pltpu.SemaphoreType.DMA,
    ],
)
def _sc_gather(px_hbm, py_hbm, pz_hbm, ptab_hbm, ftab_hbm, out_hbm,
               pbuf, idx2, w2, *rest):
    rows2 = rest[0:12]
    idx3, w3 = rest[12], rest[13]
    rows3 = rest[14:22]
    fbuf, sem = rest[22], rest[23]
    wid = lax.axis_index("s") * 2 + lax.axis_index("c")
    base_w = wid * P_PER_W

    @pl.loop(0, N_CHUNK)
    def _chunk(ci):
        off = base_w + ci * CHUNK
        pltpu.sync_copy(px_hbm.at[pl.ds(off, CHUNK)], pbuf.at[0])
        pltpu.sync_copy(py_hbm.at[pl.ds(off, CHUNK)], pbuf.at[1])
        pltpu.sync_copy(pz_hbm.at[pl.ds(off, CHUNK)], pbuf.at[2])

        @pl.loop(0, GRP)
        def _p1(g):
            s = g * LANES
            pv = [pbuf[a, pl.ds(s, LANES)] for a in range(3)]
            for p, (ia, ib) in enumerate(((0, 1), (0, 2), (1, 2))):
                u = jnp.clip((pv[ia] + 1.0) * (0.5 * (RES_P - 1)), 0.0, RES_P - 1.0)
                v = jnp.clip((pv[ib] + 1.0) * (0.5 * (RES_P - 1)), 0.0, RES_P - 1.0)
                x0 = jnp.minimum(u.astype(jnp.int32), RES_P - 2)
                y0 = jnp.minimum(v.astype(jnp.int32), RES_P - 2)
                wx = u - x0.astype(jnp.float32)
                wy = v - y0.astype(jnp.float32)
                xi = y0 * RES_P + x0 + p * (RES_P * RES_P)
                idx2[4 * p + 0, pl.ds(s, LANES)] = xi
                idx2[4 * p + 1, pl.ds(s, LANES)] = xi + 1
                idx2[4 * p + 2, pl.ds(s, LANES)] = xi + RES_P
                idx2[4 * p + 3, pl.ds(s, LANES)] = xi + RES_P + 1
                w2[4 * p + 0, pl.ds(s, LANES)] = (1.0 - wy) * (1.0 - wx)
                w2[4 * p + 1, pl.ds(s, LANES)] = (1.0 - wy) * wx
                w2[4 * p + 2, pl.ds(s, LANES)] = wy * (1.0 - wx)
                w2[4 * p + 3, pl.ds(s, LANES)] = wy * wx

        hs = [pltpu.async_copy(ptab_hbm.at[idx2.at[j]], rows2[j], sem)
              for j in range(12)]
        for h in hs:
            h.wait()

        lane = lax.iota(jnp.int32, LANES)

        @pl.loop(0, GRP)
        def _p3(g):
            s = g * LANES
            wrows = [w2[j, pl.ds(s, LANES)] for j in range(12)]
            cx = jnp.zeros((LANES,), jnp.float32)
            cy = jnp.zeros((LANES,), jnp.float32)
            cz = jnp.zeros((LANES,), jnp.float32)
            for e in range(LANES):
                i = s + e
                pls_v = []
                for p in range(3):
                    bl = None
                    for c4 in range(4):
                        j = 4 * p + c4
                        term = wrows[j][e] * rows2[j][i, :]
                        bl = term if bl is None else bl + term
                    pls_v.append(bl)
                prod = pls_v[0] * pls_v[1] * pls_v[2]
                cxe = (prod[0] + prod[1]) + (prod[2] + prod[3])
                cye = (prod[4] + prod[5]) + (prod[6] + prod[7])
                cze = (prod[8] + prod[9]) + (prod[10] + prod[11])
                m = lane == e
                cx = jnp.where(m, cxe, cx)
                cy = jnp.where(m, cye, cy)
                cz = jnp.where(m, cze, cz)

            fx = jnp.clip((cx + 1.0) * (0.5 * (RES_F - 1)), 0.0, RES_F - 1.0)
            fy = jnp.clip((cy + 1.0) * (0.5 * (RES_F - 1)), 0.0, RES_F - 1.0)
            fz = jnp.clip((cz + 1.0) * (0.5 * (RES_F - 1)), 0.0, RES_F - 1.0)
            x0 = jnp.minimum(fx.astype(jnp.int32), RES_F - 2)
            y0 = jnp.minimum(fy.astype(jnp.int32), RES_F - 2)
            z0 = jnp.minimum(fz.astype(jnp.int32), RES_F - 2)
            wx = fx - x0.astype(jnp.float32)
            wy = fy - y0.astype(jnp.float32)
            wz = fz - z0.astype(jnp.float32)
            base = (z0 * RES_F + y0) * RES_F + x0
            wxs = (1.0 - wx, wx)
            wys = (1.0 - wy, wy)
            wzs = (1.0 - wz, wz)
            for k in range(8):
                bz, by, bx = (k >> 2) & 1, (k >> 1) & 1, k & 1
                idx3[k, pl.ds(s, LANES)] = base + (bz * RES_F * RES_F + by * RES_F + bx)
                w3[k, pl.ds(s, LANES)] = wzs[bz] * wys[by] * wxs[bx]

        hs2 = [pltpu.async_copy(ftab_hbm.at[idx3.at[k]], rows3[k], sem)
               for k in range(8)]
        for h in hs2:
            h.wait()

        @pl.loop(0, GRP)
        def _p5(g):
            s = g * LANES
            wrows = [w3[k, pl.ds(s, LANES)] for k in range(8)]
            for e in range(LANES):
                i = s + e
                acc0 = None
                acc1 = None
                for k in range(8):
                    w = wrows[k][e]
                    t0 = w * rows3[k][i, pl.ds(0, LANES)]
                    t1 = w * rows3[k][i, pl.ds(LANES, LANES)]
                    acc0 = t0 if acc0 is None else acc0 + t0
                    acc1 = t1 if acc1 is None else acc1 + t1
                fbuf[i, pl.ds(0, LANES)] = acc0
                fbuf[i, pl.ds(LANES, LANES)] = acc1

        pltpu.sync_copy(fbuf, out_hbm.at[pl.ds(off, CHUNK)])


def _render_body(feats_ref, d_ref, dl_ref, bg_ref,
                 ws1a_ref, ws1b_ref, bs1_ref, ws2_ref, bs2_ref,
                 wc1a_ref, wc1b_ref, bc1_ref, wc2_ref, bc2_ref, out_ref):
    x = feats_ref[...]
    d = d_ref[...]

    h_s = jnp.dot(x, ws1a_ref[...], preferred_element_type=jnp.float32)
    h_s = h_s.reshape(RB, N_SAMPLES, WIDTH)
    h_s = h_s + jnp.dot(d, ws1b_ref[...], preferred_element_type=jnp.float32)[:, None, :]
    h_s = jnp.maximum(h_s + bs1_ref[...][None, :, :], 0.0)
    sigma = jnp.dot(h_s.reshape(RB * N_SAMPLES, WIDTH), ws2_ref[...],
                    preferred_element_type=jnp.float32)
    sigma = sigma.reshape(RB, N_SAMPLES) + bs2_ref[0, 0]
    density = jnp.exp(jnp.clip(sigma, -15.0, 15.0))

    h_c = jnp.dot(x, wc1a_ref[...], preferred_element_type=jnp.float32)
    h_c = h_c.reshape(RB, N_SAMPLES, WIDTH)
    h_c = h_c + jnp.dot(d, wc1b_ref[...], preferred_element_type=jnp.float32)[:, None, :]
    h_c = jnp.maximum(h_c + bc1_ref[...][None, :, :], 0.0)
    rgbl = jnp.dot(h_c.reshape(RB * N_SAMPLES, WIDTH), wc2_ref[...],
                   preferred_element_type=jnp.float32)
    rgbl = rgbl + bc2_ref[...]
    rgb = 1.0 / (1.0 + jnp.exp(-rgbl))
    rgb = rgb.reshape(RB, N_SAMPLES, 3)

    tau = density * dl_ref[...]
    tri = (lax.broadcasted_iota(jnp.int32, (N_SAMPLES, N_SAMPLES), 0)
           < lax.broadcasted_iota(jnp.int32, (N_SAMPLES, N_SAMPLES), 1)
           ).astype(jnp.float32)
    ctau_ex = jnp.dot(tau, tri, preferred_element_type=jnp.float32)
    trans = jnp.exp(-ctau_ex)
    alphas = 1.0 - jnp.exp(-tau)
    w = trans * alphas
    ray_colors = jnp.sum(w[:, :, None] * rgb, axis=1)
    alpha = jnp.sum(w, axis=1)
    out_ref[...] = ray_colors + (1.0 - alpha)[:, None] * bg_ref[...]


_render = pl.pallas_call(
    _render_body,
    grid=(N_RB,),
    in_specs=[
        pl.BlockSpec((RB * N_SAMPLES, FEAT), lambda i: (i, 0)),
        pl.BlockSpec((RB, 3), lambda i: (i, 0)),
        pl.BlockSpec((RB, 1), lambda i: (i, 0)),
        pl.BlockSpec((RB, 3), lambda i: (i, 0)),
        pl.BlockSpec((FEAT, WIDTH), lambda i: (0, 0)),
        pl.BlockSpec((3, WIDTH), lambda i: (0, 0)),
        pl.BlockSpec((1, WIDTH), lambda i: (0, 0)),
        pl.BlockSpec((WIDTH, 1), lambda i: (0, 0)),
        pl.BlockSpec((1, 1), lambda i: (0, 0)),
        pl.BlockSpec((FEAT, WIDTH), lambda i: (0, 0)),
        pl.BlockSpec((3, WIDTH), lambda i: (0, 0)),
        pl.BlockSpec((1, WIDTH), lambda i: (0, 0)),
        pl.BlockSpec((WIDTH, 3), lambda i: (0, 0)),
        pl.BlockSpec((1, 3), lambda i: (0, 0)),
    ],
    out_specs=pl.BlockSpec((RB, 3), lambda i: (i, 0)),
    out_shape=jax.ShapeDtypeStruct((N_RAYS, 3), jnp.float32),
)


def kernel(rays_o, rays_d, bg_color, grid0, features,
           w_s1, b_s1, w_s2, b_s2, w_c1, b_c1, w_c2, b_c2):
    px, py, pz, d_t, deltas = _ray_setup(rays_o.T, rays_d.T)

    ptab = jnp.pad(jnp.transpose(grid0, (0, 2, 3, 1)),
                   ((0, 0), (0, 0), (0, 0), (0, 4))).reshape(3 * RES_P * RES_P, 16)
    ftab = jnp.transpose(features, (1, 2, 3, 0)).reshape(RES_F ** 3, FEAT)

    feats = _sc_gather(px.reshape(-1), py.reshape(-1), pz.reshape(-1), ptab, ftab)

    return _render(feats, d_t.T, deltas.reshape(N_RAYS, 1), bg_color,
                   w_s1[:FEAT], w_s1[FEAT:], b_s1.reshape(1, WIDTH), w_s2,
                   b_s2.reshape(1, 1),
                   w_c1[:FEAT], w_c1[FEAT:], b_c1.reshape(1, WIDTH), w_c2,
                   b_c2.reshape(1, 3))
